# baseline (speedup 1.0000x reference)
import functools

import jax
import jax.numpy as jnp
from jax import lax
from jax.experimental import pallas as pl
from jax.experimental.pallas import tpu as pltpu
from jax.experimental.pallas import tpu_sc as plsc

EMBED_DIM = 64
NC = 2
NS = 16
NW = NC * NS

_RPC = 2
_NBUF = 2


@functools.partial(jax.jit, static_argnums=(2, 3))
def _embed_lookup(table, idx, N, S):
    rows_w = N // NW
    n_chunks = rows_w // _RPC
    n_outer = n_chunks // _NBUF
    s_lo = 128
    s_hi = S - s_lo

    mesh = plsc.VectorSubcoreMesh(core_axis_name="c", subcore_axis_name="s")

    @functools.partial(
        pl.kernel,
        mesh=mesh,
        out_type=jax.ShapeDtypeStruct((N, S, EMBED_DIM), jnp.float32),
        scratch_types=[
            pltpu.VMEM((rows_w, S), jnp.int32),
            pltpu.VMEM((_NBUF, _RPC, S, EMBED_DIM), jnp.float32),
            pltpu.SemaphoreType.DMA((_NBUF,)),
            pltpu.SemaphoreType.DMA((_NBUF,)),
        ],
        compiler_params=pltpu.CompilerParams(use_tc_tiling_on_sc=False),
    )
    def embed_kernel(table_hbm, idx_hbm, out_hbm, idx_all, rows_v, gsem, ssem):
        wid = lax.axis_index("s") * NC + lax.axis_index("c")
        row_base = wid * rows_w
        pltpu.sync_copy(idx_hbm.at[pl.ds(row_base, rows_w)], idx_all)

        def fire_gather(i, b):
            for r in range(_RPC):
                row = i * _RPC + r
                pltpu.async_copy(
                    table_hbm.at[idx_all.at[row].at[pl.ds(0, s_lo)]],
                    rows_v.at[b].at[r].at[pl.ds(0, s_lo)],
                    gsem.at[b],
                )
                pltpu.async_copy(
                    table_hbm.at[idx_all.at[row].at[pl.ds(s_lo, s_hi)]],
                    rows_v.at[b].at[r].at[pl.ds(s_lo, s_hi)],
                    gsem.at[b],
                )

        def wait_gather(b):
            pltpu.make_async_copy(
                out_hbm.at[pl.ds(0, _RPC)], rows_v.at[b], gsem.at[b]
            ).wait()

        def fire_store(i, b):
            start = pl.multiple_of(row_base + i * _RPC, _RPC)
            pltpu.async_copy(
                rows_v.at[b], out_hbm.at[pl.ds(start, _RPC)], ssem.at[b]
            )

        def wait_store(b):
            pltpu.make_async_copy(
                out_hbm.at[pl.ds(0, _RPC)], rows_v.at[b], ssem.at[b]
            ).wait()

        def outer(o, carry):
            for b in range(_NBUF):
                i = o * _NBUF + b

                @pl.when(o > 0)
                def _():
                    wait_store(b)

                fire_gather(i, b)
                pb = (b - 1) % _NBUF
                if b == 0:
                    @pl.when(o > 0)
                    def _():
                        wait_gather(pb)
                        fire_store(i - 1, pb)
                else:
                    wait_gather(pb)
                    fire_store(i - 1, pb)
            return carry

        lax.fori_loop(0, n_outer, outer, 0)
        last = n_chunks - 1
        wait_gather(_NBUF - 1)
        fire_store(last, _NBUF - 1)
        for b in range(_NBUF):
            wait_store(b)

    return embed_kernel(table, idx)


def kernel(pre_embedding, preembed_mask, embed_table):
    N, S = pre_embedding.shape
    out = _embed_lookup(embed_table, pre_embedding, N, S)
    return out, preembed_mask
